# baseline (speedup 1.0000x reference)
import functools
import math

import jax
import jax.numpy as jnp
from jax import lax
from jax.experimental import pallas as pl
from jax.experimental.pallas import tpu as pltpu
from jax.experimental.pallas import tpu_sc as plsc

N = 1000000
M = 20000
NW = 32
BM = 640
SW = 320
MPAD = NW * BM // 2 * 2
NQ = 18
T = 2000
GROUPS = T // 16


def _make_sc_body(a_lo, a_hi):
  NT = (a_hi - a_lo) // T
  NTP = ((NT + 127) // 128) * 128

  def _sc_body(x1_hbm, y1_hbm, z1_hbm, x2_hbm, y2_hbm, z2_hbm,
               ids_hbm, out_hbm,
               heads_i, heads_v, ids_a, x1_a, y1_a, z1_a, x2_a, y2_a, z2_a,
               ids_b, x1_b, y1_b, z1_b, x2_b, y2_b, z2_b,
               acc_v, sem_a, sem_b):
    wid = lax.axis_index("c") * 16 + lax.axis_index("s")

    iota16 = lax.iota(jnp.int32, 16)

    def _hidx(j, _):
        heads_i[pl.ds(j * 16, 16)] = jnp.minimum(
            a_lo + (j * 16 + iota16) * T, a_hi - 8)
        return 0

    lax.fori_loop(0, NTP // 16, _hidx, 0)
    for cph in range(NTP // 128):
        pltpu.async_copy(ids_hbm.at[heads_i.at[pl.ds(cph * 128, 128)]],
                         heads_v.at[pl.ds(cph * 128, 128)], sem_a)
    for cph in range(NTP // 128):
        pltpu.make_async_copy(
            ids_hbm.at[heads_i.at[pl.ds(cph * 128, 128)]],
            heads_v.at[pl.ds(cph * 128, 128)], sem_a).wait()

    zeros16 = jnp.zeros((16,), jnp.float32)

    def _zero(j, _):
        acc_v[pl.ds(j * 16, 16)] = zeros16
        return 0

    lax.fori_loop(0, (NQ * BM) // 16 + 1, _zero, 0)

    ones16 = jnp.ones((16,), jnp.float32)
    lane_off = lax.iota(jnp.int32, 16) * (T // 16)

    hbms = (ids_hbm, x1_hbm, y1_hbm, z1_hbm, x2_hbm, y2_hbm, z2_hbm)
    bufs = ((ids_a, x1_a, y1_a, z1_a, x2_a, y2_a, z2_a),
            (ids_b, x1_b, y1_b, z1_b, x2_b, y2_b, z2_b))
    sems = (sem_a, sem_b)

    def issue(k, b):
        a0 = k * T
        pltpu.async_copy(ids_hbm.at[pl.ds(a_lo + a0, T)], bufs[b][0],
                         sems[b])
        for h, v in zip(hbms[1:], bufs[b][1:]):
            pltpu.async_copy(h.at[pl.ds(a0, T)], v, sems[b])

    def drain(b):
        for h, v in zip(hbms, bufs[b]):
            pltpu.make_async_copy(h.at[pl.ds(0, T)], v, sems[b]).wait()

    for half in range(2):
        base_id = (wid + 32 * half) * SW
        row_off = half * SW

        def _count(j, c):
            h = heads_v[pl.ds(j * 16, 16)]
            valid = (j * 16 + iota16) < NT
            c0 = plsc.all_reduce_population_count((h < base_id) & valid)
            c1 = plsc.all_reduce_population_count((h < base_id + SW) & valid)
            return (c[0] + c0[0], c[1] + c1[0])

        cnt0, cnt1 = lax.fori_loop(0, NTP // 16, _count, (0, 0))
        k0 = jnp.maximum(cnt0 - 1, 0)
        k1 = jnp.minimum(cnt1, NT)

        def process(b):
            ids_v, x1_v, y1_v, z1_v, x2_v, y2_v, z2_v = bufs[b]

            def _group(i, _):
                a = lane_off + i
                ids16 = plsc.load_gather(ids_v, [a])
                lid = ids16 - base_id
                msk = (lid >= 0) & (lid < SW)
                base = jnp.where(msk, lid + row_off, NQ * BM)
                x1 = plsc.load_gather(x1_v, [a])
                y1 = plsc.load_gather(y1_v, [a])
                z1 = plsc.load_gather(z1_v, [a])
                x2 = plsc.load_gather(x2_v, [a])
                y2 = plsc.load_gather(y2_v, [a])
                z2 = plsc.load_gather(z2_v, [a])
                q1 = x1 * x1 + y1 * y1 + z1 * z1
                q2 = x2 * x2 + y2 * y2 + z2 * z2
                vals = (ones16, x1, y1, z1, x2, y2, z2, q1, q2,
                        x2 * x1, x2 * y1, x2 * z1,
                        y2 * x1, y2 * y1, y2 * z1,
                        z2 * x1, z2 * y1, z2 * z1)
                for q, v in enumerate(vals):
                    plsc.addupdate_scatter(acc_v, [base + q * BM], v,
                                           mask=msk)
                return 0

            lax.fori_loop(0, GROUPS, _group, 0)

        @pl.when(k0 < k1)
        def _():
            issue(k0, 0)

        @pl.when(k0 + 1 < k1)
        def _():
            issue(k0 + 1, 1)

        def _pair(p, _):
            for b in range(2):
                k = k0 + 2 * p + b

                @pl.when(k < k1)
                def _():
                    drain(b)
                    process(b)

                    @pl.when(k + 2 < k1)
                    def _():
                        issue(k + 2, b)
            return 0

        lax.fori_loop(0, (k1 - k0 + 1) // 2, _pair, 0)

    for q in range(NQ):
        for half in range(2):
            pltpu.sync_copy(
                acc_v.at[pl.ds(q * BM + half * SW, SW)],
                out_hbm.at[pl.ds(q * MPAD + (wid + 32 * half) * SW, SW)])

  return _sc_body, NTP


def _sc_sums(x1, y1, z1, x2, y2, z2, ids, a_lo):
    body, NTP = _make_sc_body(a_lo, a_lo + x1.shape[0])
    mesh = plsc.VectorSubcoreMesh(core_axis_name="c", subcore_axis_name="s",
                                  num_cores=2, num_subcores=16)
    f = pl.kernel(
        body,
        out_type=jax.ShapeDtypeStruct((NQ * MPAD,), jnp.float32),
        mesh=mesh,
        scratch_types=[
            pltpu.VMEM((NTP,), jnp.int32),
            pltpu.VMEM((NTP,), jnp.int32),
            pltpu.VMEM((T,), jnp.int32),
        ] + [pltpu.VMEM((T,), jnp.float32)] * 6 + [
            pltpu.VMEM((T,), jnp.int32),
        ] + [pltpu.VMEM((T,), jnp.float32)] * 6 + [
            pltpu.VMEM((2 * NQ * BM,), jnp.float32),
            pltpu.SemaphoreType.DMA,
            pltpu.SemaphoreType.DMA,
        ],
        compiler_params=pltpu.CompilerParams(needs_layout_passes=False),
    )
    return f(x1, y1, z1, x2, y2, z2, ids)


def _kabsch_body(s_ref, o_ref):
    g = lambda q: s_ref[q]
    cnt = g(0)
    n = jnp.maximum(cnt, 1.0)
    ninv = 1.0 / n
    m1x, m1y, m1z = g(1) * ninv, g(2) * ninv, g(3) * ninv
    m2x, m2y, m2z = g(4) * ninv, g(5) * ninv, g(6) * ninv
    q1, q2 = g(7), g(8)
    m1 = (m1x, m1y, m1z)
    m2 = (m2x, m2y, m2z)
    c = [[g(9 + 3 * i + j) * ninv - m2[i] * m1[j] for j in range(3)]
         for i in range(3)]
    det = (c[0][0] * (c[1][1] * c[2][2] - c[1][2] * c[2][1])
           - c[0][1] * (c[1][0] * c[2][2] - c[1][2] * c[2][0])
           + c[0][2] * (c[1][0] * c[2][1] - c[1][1] * c[2][0]))
    def B(a, b):
        return c[0][a] * c[0][b] + c[1][a] * c[1][b] + c[2][a] * c[2][b]
    b00, b11, b22 = B(0, 0), B(1, 1), B(2, 2)
    b01, b02, b12 = B(0, 1), B(0, 2), B(1, 2)
    q = (b00 + b11 + b22) * (1.0 / 3.0)
    p1 = b01 * b01 + b02 * b02 + b12 * b12
    d0, d1, d2 = b00 - q, b11 - q, b22 - q
    p2 = d0 * d0 + d1 * d1 + d2 * d2 + 2.0 * p1
    p = jnp.sqrt(jnp.maximum(p2 * (1.0 / 6.0), 0.0))
    pinv = 1.0 / jnp.maximum(p, 1e-30)
    e00, e11, e22 = d0 * pinv, d1 * pinv, d2 * pinv
    e01, e02, e12 = b01 * pinv, b02 * pinv, b12 * pinv
    detC = (e00 * (e11 * e22 - e12 * e12)
            - e01 * (e01 * e22 - e12 * e02)
            + e02 * (e01 * e12 - e11 * e02))
    r = jnp.clip(detC * 0.5, -1.0, 1.0)
    acos_r = jnp.arctan2(jnp.sqrt(jnp.maximum(1.0 - r * r, 0.0)), r)
    phi = acos_r * (1.0 / 3.0)
    eig1 = q + 2.0 * p * jnp.cos(phi)
    eig3 = q + 2.0 * p * jnp.cos(phi + (2.0 * math.pi / 3.0))
    eig2 = 3.0 * q - eig1 - eig3
    s1 = jnp.sqrt(jnp.maximum(eig1, 0.0))
    s2 = jnp.sqrt(jnp.maximum(eig2, 0.0))
    s3 = jnp.sqrt(jnp.maximum(eig3, 0.0))
    ssum = s1 + s2 + s3
    smin = jnp.minimum(jnp.minimum(s1, s2), s3)
    ssum = jnp.where(det < 0.0, ssum - 2.0 * smin, ssum)
    sigma = ssum * (1.0 / 3.0)
    var1 = (q1 * ninv - (m1x * m1x + m1y * m1y + m1z * m1z)) * (1.0 / 3.0)
    var2 = (q2 * ninv - (m2x * m2x + m2y * m2y + m2z * m2z)) * (1.0 / 3.0)
    o_ref[...] = var1 + var2 - 2.0 * sigma


def _kabsch(sums3d):
    return pl.pallas_call(
        _kabsch_body,
        out_shape=jax.ShapeDtypeStruct((MPAD // 128, 128), jnp.float32),
    )(sums3d)


def kernel(coordinates1, coordinates2, molecule_ix):
    ids = molecule_ix
    cuts = (0, 256000, 512000, 768000, N)
    sums = None
    for lo, hi in zip(cuts[:-1], cuts[1:]):
        s = _sc_sums(
            coordinates1[lo:hi, 0], coordinates1[lo:hi, 1],
            coordinates1[lo:hi, 2],
            coordinates2[lo:hi, 0], coordinates2[lo:hi, 1],
            coordinates2[lo:hi, 2],
            ids, lo)
        sums = s if sums is None else sums + s
    out = _kabsch(sums.reshape(NQ, MPAD // 128, 128))
    return out.reshape(-1)[:M]


# speedup vs baseline: 1.6401x; 1.6401x over previous
import functools
import math

import jax
import jax.numpy as jnp
from jax import lax
from jax.experimental import pallas as pl
from jax.experimental.pallas import tpu as pltpu
from jax.experimental.pallas import tpu_sc as plsc

N = 1000000
M = 20000
NW = 32
BM = 640
SW = 320
MPAD = NW * BM // 2 * 2
NQ = 18
T = 2000
GROUPS = T // 16


def _make_sc_body(a_lo, a_hi):
  NT = (a_hi - a_lo) // T
  NTP = ((NT + 127) // 128) * 128

  def _sc_body(x1_hbm, y1_hbm, z1_hbm, x2_hbm, y2_hbm, z2_hbm,
               ids_hbm, out_hbm,
               heads_i, heads_v, ids_a, x1_a, y1_a, z1_a, x2_a, y2_a, z2_a,
               ids_b, x1_b, y1_b, z1_b, x2_b, y2_b, z2_b,
               acc_v, sem_a, sem_b):
    wid = lax.axis_index("c") * 16 + lax.axis_index("s")

    iota16 = lax.iota(jnp.int32, 16)

    def _hidx(j, _):
        heads_i[pl.ds(j * 16, 16)] = jnp.minimum(
            a_lo + (j * 16 + iota16) * T, a_hi - 8)
        return 0

    lax.fori_loop(0, NTP // 16, _hidx, 0)
    for cph in range(NTP // 128):
        pltpu.async_copy(ids_hbm.at[heads_i.at[pl.ds(cph * 128, 128)]],
                         heads_v.at[pl.ds(cph * 128, 128)], sem_a)
    for cph in range(NTP // 128):
        pltpu.make_async_copy(
            ids_hbm.at[heads_i.at[pl.ds(cph * 128, 128)]],
            heads_v.at[pl.ds(cph * 128, 128)], sem_a).wait()

    zeros16 = jnp.zeros((16,), jnp.float32)

    def _zero(j, _):
        acc_v[pl.ds(j * 16, 16)] = zeros16
        return 0

    lax.fori_loop(0, (NQ * BM) // 16 + 1, _zero, 0)

    ones16 = jnp.ones((16,), jnp.float32)
    lane_off = lax.iota(jnp.int32, 16) * (T // 16)

    hbms = (ids_hbm, x1_hbm, y1_hbm, z1_hbm, x2_hbm, y2_hbm, z2_hbm)
    bufs = ((ids_a, x1_a, y1_a, z1_a, x2_a, y2_a, z2_a),
            (ids_b, x1_b, y1_b, z1_b, x2_b, y2_b, z2_b))
    sems = (sem_a, sem_b)

    def issue(k, b):
        a0 = k * T
        pltpu.async_copy(ids_hbm.at[pl.ds(a_lo + a0, T)], bufs[b][0],
                         sems[b])
        for h, v in zip(hbms[1:], bufs[b][1:]):
            pltpu.async_copy(h.at[pl.ds(a0, T)], v, sems[b])

    def drain(b):
        for h, v in zip(hbms, bufs[b]):
            pltpu.make_async_copy(h.at[pl.ds(0, T)], v, sems[b]).wait()

    for half in range(2):
        base_id = (wid + 32 * half) * SW
        row_off = half * SW

        def _count(j, c):
            h = heads_v[pl.ds(j * 16, 16)]
            valid = (j * 16 + iota16) < NT
            c0 = plsc.all_reduce_population_count((h < base_id) & valid)
            c1 = plsc.all_reduce_population_count((h < base_id + SW) & valid)
            return (c[0] + c0[0], c[1] + c1[0])

        cnt0, cnt1 = lax.fori_loop(0, NTP // 16, _count, (0, 0))
        k0 = jnp.maximum(cnt0 - 1, 0)
        k1 = jnp.minimum(cnt1, NT)

        def process(b):
            ids_v, x1_v, y1_v, z1_v, x2_v, y2_v, z2_v = bufs[b]

            def _group(i, _):
                a = lane_off + i
                ids16 = plsc.load_gather(ids_v, [a])
                lid = ids16 - base_id
                msk = (lid >= 0) & (lid < SW)
                base = jnp.where(msk, lid + row_off, NQ * BM)
                x1 = plsc.load_gather(x1_v, [a])
                y1 = plsc.load_gather(y1_v, [a])
                z1 = plsc.load_gather(z1_v, [a])
                x2 = plsc.load_gather(x2_v, [a])
                y2 = plsc.load_gather(y2_v, [a])
                z2 = plsc.load_gather(z2_v, [a])
                q1 = x1 * x1 + y1 * y1 + z1 * z1
                q2 = x2 * x2 + y2 * y2 + z2 * z2
                vals = (ones16, x1, y1, z1, x2, y2, z2, q1, q2,
                        x2 * x1, x2 * y1, x2 * z1,
                        y2 * x1, y2 * y1, y2 * z1,
                        z2 * x1, z2 * y1, z2 * z1)
                for q, v in enumerate(vals):
                    plsc.addupdate_scatter(acc_v, [base + q * BM], v,
                                           mask=msk)
                return 0

            lax.fori_loop(0, GROUPS, _group, 0)

        @pl.when(k0 < k1)
        def _():
            issue(k0, 0)

        @pl.when(k0 + 1 < k1)
        def _():
            issue(k0 + 1, 1)

        def _pair(p, _):
            for b in range(2):
                k = k0 + 2 * p + b

                @pl.when(k < k1)
                def _():
                    drain(b)
                    process(b)

                    @pl.when(k + 2 < k1)
                    def _():
                        issue(k + 2, b)
            return 0

        lax.fori_loop(0, (k1 - k0 + 1) // 2, _pair, 0)

    for q in range(NQ):
        for half in range(2):
            pltpu.sync_copy(
                acc_v.at[pl.ds(q * BM + half * SW, SW)],
                out_hbm.at[pl.ds(q * MPAD + (wid + 32 * half) * SW, SW)])

  return _sc_body, NTP


def _sc_sums(x1, y1, z1, x2, y2, z2, ids, a_lo):
    body, NTP = _make_sc_body(a_lo, a_lo + x1.shape[0])
    mesh = plsc.VectorSubcoreMesh(core_axis_name="c", subcore_axis_name="s",
                                  num_cores=2, num_subcores=16)
    f = pl.kernel(
        body,
        out_type=jax.ShapeDtypeStruct((NQ * MPAD,), jnp.float32),
        mesh=mesh,
        scratch_types=[
            pltpu.VMEM((NTP,), jnp.int32),
            pltpu.VMEM((NTP,), jnp.int32),
            pltpu.VMEM((T,), jnp.int32),
        ] + [pltpu.VMEM((T,), jnp.float32)] * 6 + [
            pltpu.VMEM((T,), jnp.int32),
        ] + [pltpu.VMEM((T,), jnp.float32)] * 6 + [
            pltpu.VMEM((2 * NQ * BM,), jnp.float32),
            pltpu.SemaphoreType.DMA,
            pltpu.SemaphoreType.DMA,
        ],
        compiler_params=pltpu.CompilerParams(needs_layout_passes=False),
    )
    return f(x1, y1, z1, x2, y2, z2, ids)


def _kabsch_body(s_ref, o_ref):
    g = lambda q: s_ref[q]
    cnt = g(0)
    n = jnp.maximum(cnt, 1.0)
    ninv = 1.0 / n
    m1x, m1y, m1z = g(1) * ninv, g(2) * ninv, g(3) * ninv
    m2x, m2y, m2z = g(4) * ninv, g(5) * ninv, g(6) * ninv
    q1, q2 = g(7), g(8)
    m1 = (m1x, m1y, m1z)
    m2 = (m2x, m2y, m2z)
    c = [[g(9 + 3 * i + j) * ninv - m2[i] * m1[j] for j in range(3)]
         for i in range(3)]
    det = (c[0][0] * (c[1][1] * c[2][2] - c[1][2] * c[2][1])
           - c[0][1] * (c[1][0] * c[2][2] - c[1][2] * c[2][0])
           + c[0][2] * (c[1][0] * c[2][1] - c[1][1] * c[2][0]))
    def B(a, b):
        return c[0][a] * c[0][b] + c[1][a] * c[1][b] + c[2][a] * c[2][b]
    b00, b11, b22 = B(0, 0), B(1, 1), B(2, 2)
    b01, b02, b12 = B(0, 1), B(0, 2), B(1, 2)
    q = (b00 + b11 + b22) * (1.0 / 3.0)
    p1 = b01 * b01 + b02 * b02 + b12 * b12
    d0, d1, d2 = b00 - q, b11 - q, b22 - q
    p2 = d0 * d0 + d1 * d1 + d2 * d2 + 2.0 * p1
    p = jnp.sqrt(jnp.maximum(p2 * (1.0 / 6.0), 0.0))
    pinv = 1.0 / jnp.maximum(p, 1e-30)
    e00, e11, e22 = d0 * pinv, d1 * pinv, d2 * pinv
    e01, e02, e12 = b01 * pinv, b02 * pinv, b12 * pinv
    detC = (e00 * (e11 * e22 - e12 * e12)
            - e01 * (e01 * e22 - e12 * e02)
            + e02 * (e01 * e12 - e11 * e02))
    r = jnp.clip(detC * 0.5, -1.0, 1.0)
    acos_r = jnp.arctan2(jnp.sqrt(jnp.maximum(1.0 - r * r, 0.0)), r)
    phi = acos_r * (1.0 / 3.0)
    eig1 = q + 2.0 * p * jnp.cos(phi)
    eig3 = q + 2.0 * p * jnp.cos(phi + (2.0 * math.pi / 3.0))
    eig2 = 3.0 * q - eig1 - eig3
    s1 = jnp.sqrt(jnp.maximum(eig1, 0.0))
    s2 = jnp.sqrt(jnp.maximum(eig2, 0.0))
    s3 = jnp.sqrt(jnp.maximum(eig3, 0.0))
    ssum = s1 + s2 + s3
    smin = jnp.minimum(jnp.minimum(s1, s2), s3)
    ssum = jnp.where(det < 0.0, ssum - 2.0 * smin, ssum)
    sigma = ssum * (1.0 / 3.0)
    var1 = (q1 * ninv - (m1x * m1x + m1y * m1y + m1z * m1z)) * (1.0 / 3.0)
    var2 = (q2 * ninv - (m2x * m2x + m2y * m2y + m2z * m2z)) * (1.0 / 3.0)
    o_ref[...] = var1 + var2 - 2.0 * sigma


def _kabsch(sums3d):
    return pl.pallas_call(
        _kabsch_body,
        out_shape=jax.ShapeDtypeStruct((MPAD // 128, 128), jnp.float32),
    )(sums3d)


def kernel(coordinates1, coordinates2, molecule_ix):
    ids = molecule_ix
    H = 512000
    sums1 = _sc_sums(
        coordinates1[:H, 0], coordinates1[:H, 1], coordinates1[:H, 2],
        coordinates2[:H, 0], coordinates2[:H, 1], coordinates2[:H, 2],
        ids, 0)
    sums2 = _sc_sums(
        coordinates1[H:, 0], coordinates1[H:, 1], coordinates1[H:, 2],
        coordinates2[H:, 0], coordinates2[H:, 1], coordinates2[H:, 2],
        ids, H)
    out = _kabsch((sums1 + sums2).reshape(NQ, MPAD // 128, 128))
    return out.reshape(-1)[:M]
